# baseline (speedup 1.0000x reference)
import functools

import jax
import jax.numpy as jnp
from jax import lax
from jax.experimental import pallas as pl
from jax.experimental.pallas import tpu as pltpu
from jax.experimental.pallas import tpu_sc as plsc

_NC = 2
_NS = 16
_NW = _NC * _NS
_CH = 128


def _make_gather(n_nodes, d, per_w):
  mesh = plsc.VectorSubcoreMesh(core_axis_name="c", subcore_axis_name="s")

  @functools.partial(
      pl.kernel,
      mesh=mesh,
      out_type=jax.ShapeDtypeStruct((_NW * per_w * _CH, d), jnp.float32),
      scratch_types=[
          pltpu.VMEM((per_w, _CH), jnp.int32),
          pltpu.VMEM((per_w * _CH, d), jnp.float32),
          pltpu.VMEM_SHARED((n_nodes, d), jnp.float32),
          pltpu.SemaphoreType.DMA,
      ],
      compiler_params=pltpu.CompilerParams(use_tc_tiling_on_sc=False),
  )
  def k(table, idx_hbm, out, idx_v, rows_v, tbl_s, sem):
    sid = lax.axis_index("s")
    w = sid * _NC + lax.axis_index("c")

    @pl.when(sid == 0)
    def _():
      pltpu.sync_copy(table, tbl_s)

    pltpu.sync_copy(idx_hbm.at[w], idx_v)
    plsc.subcore_barrier()
    copies = []
    for j in range(per_w):
      copies.append(pltpu.async_copy(
          tbl_s.at[idx_v.at[j]], rows_v.at[pl.ds(j * _CH, _CH)], sem))
    for c in copies:
      c.wait()
    pltpu.sync_copy(rows_v, out.at[pl.ds(w * per_w * _CH, per_w * _CH)])

  return k


def _make_scatter(n_nodes, d, per_w):
  mesh = plsc.VectorSubcoreMesh(core_axis_name="c", subcore_axis_name="s")

  @functools.partial(
      pl.kernel,
      mesh=mesh,
      out_type=jax.ShapeDtypeStruct((_NC, n_nodes, d), jnp.float32),
      scratch_types=[
          pltpu.VMEM((per_w, _CH), jnp.int32),
          pltpu.VMEM((per_w * _CH, d), jnp.float32),
          pltpu.VMEM_SHARED((n_nodes, d), jnp.float32),
      ],
      compiler_params=pltpu.CompilerParams(use_tc_tiling_on_sc=False),
  )
  def k(vals_hbm, idx_hbm, zeros_hbm, out, idx_v, rows_v, acc):
    cid = lax.axis_index("c")
    sid = lax.axis_index("s")
    w = cid * _NS + sid

    @pl.when(sid == 0)
    def _():
      pltpu.sync_copy(zeros_hbm, acc)

    plsc.subcore_barrier()
    pltpu.sync_copy(idx_hbm.at[w], idx_v)
    pltpu.sync_copy(vals_hbm.at[pl.ds(w * per_w * _CH, per_w * _CH)], rows_v)
    for j in range(per_w):
      pltpu.sync_copy(rows_v.at[pl.ds(j * _CH, _CH)],
                      acc.at[idx_v.at[j]], add=True)
    plsc.subcore_barrier()

    @pl.when(sid == 0)
    def _():
      pltpu.sync_copy(acc, out.at[cid])

  return k


def _edge_math_body(g_ref, aux_ref, w_all_ref, w_comp_ref,
                    slw_ref, a_wt_ref, a_b_ref, b_wt_ref, b_b_ref, out_ref):
  f32 = jnp.float32
  gb = g_ref[...]
  aux = aux_ref[...]
  blk = gb.shape[0]
  n_rels, n_bases = w_comp_ref.shape
  d_in = gb.shape[1] // 2
  d_out = slw_ref.shape[1]
  na = a_wt_ref.shape[0] - gb.shape[1]
  relf = aux[:, na:na + 1]
  onehot = (relf == lax.broadcasted_iota(
      jnp.int32, (1, n_rels), 1).astype(f32)).astype(f32)
  coeff = jnp.dot(onehot, w_comp_ref[...], preferred_element_type=f32)
  src = gb[:, :d_in]
  tgt = gb[:, d_in:]
  y = jnp.dot(src, w_all_ref[...], preferred_element_type=f32)
  msg = jnp.zeros((blk, d_out), dtype=f32)
  for b in range(n_bases):
    msg = msg + coeff[:, b:b + 1] * y[:, b * d_out:(b + 1) * d_out]
  ecat = jnp.concatenate([gb, aux[:, :na]], axis=1)
  h = jnp.maximum(
      jnp.dot(ecat, a_wt_ref[...], preferred_element_type=f32) + a_b_ref[...],
      0.0)
  logit = jnp.dot(h, b_wt_ref[...], preferred_element_type=f32) + b_b_ref[...]
  att = 1.0 / (1.0 + jnp.exp(-logit))
  curr = jnp.dot(tgt, slw_ref[...], preferred_element_type=f32) + msg * att
  half = blk // 2
  out_ref[...] = jnp.concatenate([curr[:half], curr[half:]], axis=1)


def _combine_body(p_ref, o_ref):
  o_ref[...] = p_ref[0] + p_ref[1]


def kernel(node_feat, e2n_sp, total_target_relation, total_edge,
           total_relation_embed, total_relation, weight, w_comp,
           self_loop_weight, A_w, A_b, B_w, B_b):
  del e2n_sp
  n_nodes, inp_dim = node_feat.shape
  n_edges = total_edge.shape[1]
  out_dim = self_loop_weight.shape[1]
  attn_dim = total_relation_embed.shape[1]
  n_bases = weight.shape[0]
  f32 = jnp.float32

  grain = _NW * _CH
  e_pad = -(-n_edges // grain) * grain
  pad = e_pad - n_edges
  n_acc = n_nodes + 16

  per_w_g = 2 * e_pad // grain
  idx_int = jnp.stack(
      [jnp.pad(total_edge[0], (0, pad)), jnp.pad(total_edge[1], (0, pad))],
      axis=1).reshape(_NW, per_w_g, _CH)
  g = _make_gather(n_nodes, inp_dim, per_w_g)(node_feat, idx_int)
  gp = g.reshape(e_pad, 2 * inp_dim)

  aux = jnp.concatenate(
      [total_relation_embed, total_target_relation,
       total_relation.astype(f32)[:, None],
       jnp.zeros((n_edges, 2 * inp_dim - 2 * attn_dim - 1), f32)], axis=1)
  aux = jnp.pad(aux, ((0, pad), (0, 0)))

  blk = 4096
  n_blk = e_pad // blk
  w_all = jnp.transpose(weight, (1, 0, 2)).reshape(inp_dim, n_bases * out_dim)
  curr2 = pl.pallas_call(
      _edge_math_body,
      grid=(n_blk,),
      in_specs=[
          pl.BlockSpec((blk, 2 * inp_dim), lambda i: (i, 0)),
          pl.BlockSpec((blk, 2 * inp_dim), lambda i: (i, 0)),
          pl.BlockSpec(w_all.shape, lambda i: (0, 0)),
          pl.BlockSpec(w_comp.shape, lambda i: (0, 0)),
          pl.BlockSpec(self_loop_weight.shape, lambda i: (0, 0)),
          pl.BlockSpec(A_w.shape[::-1], lambda i: (0, 0)),
          pl.BlockSpec((1, A_b.shape[0]), lambda i: (0, 0)),
          pl.BlockSpec(B_w.shape[::-1], lambda i: (0, 0)),
          pl.BlockSpec((1, 1), lambda i: (0, 0)),
      ],
      out_specs=pl.BlockSpec((blk // 2, 2 * out_dim), lambda i: (i, 0)),
      out_shape=jax.ShapeDtypeStruct((e_pad // 2, 2 * out_dim), f32),
  )(gp, aux, w_all, w_comp,
    self_loop_weight, A_w.T, A_b.reshape(1, -1), B_w.T, B_b.reshape(1, 1))
  curr = curr2.reshape(e_pad, out_dim)

  per_w_s = e_pad // grain
  idx_t = jnp.pad(total_edge[1], (0, pad), constant_values=n_nodes).reshape(
      e_pad // blk, 2, blk // 2).transpose(0, 2, 1).reshape(
      _NW, per_w_s, _CH)
  partials = _make_scatter(n_acc, out_dim, per_w_s)(
      curr, idx_t, jnp.zeros((n_acc, out_dim), f32))

  summed = pl.pallas_call(
      _combine_body,
      grid=(1,),
      in_specs=[
          pl.BlockSpec((_NC, n_acc // 2, 2 * out_dim), lambda i: (0, 0, 0))],
      out_specs=pl.BlockSpec((n_acc // 2, 2 * out_dim), lambda i: (0, 0)),
      out_shape=jax.ShapeDtypeStruct((n_acc // 2, 2 * out_dim), f32),
  )(partials.reshape(_NC, n_acc // 2, 2 * out_dim))
  return summed.reshape(n_acc, out_dim)[:n_nodes]


# speedup vs baseline: 1.0111x; 1.0111x over previous
import functools

import jax
import jax.numpy as jnp
from jax import lax
from jax.experimental import pallas as pl
from jax.experimental.pallas import tpu as pltpu
from jax.experimental.pallas import tpu_sc as plsc

_NC = 2
_NS = 16
_NW = _NC * _NS
_CH = 128


def _make_gather(n_nodes, d, per_w):
  mesh = plsc.VectorSubcoreMesh(core_axis_name="c", subcore_axis_name="s")

  @functools.partial(
      pl.kernel,
      mesh=mesh,
      out_type=jax.ShapeDtypeStruct((_NW * per_w * _CH, d), jnp.float32),
      scratch_types=[
          pltpu.VMEM((per_w, _CH), jnp.int32),
          pltpu.VMEM((per_w * _CH, d), jnp.float32),
          pltpu.VMEM_SHARED((n_nodes, d), jnp.float32),
          pltpu.SemaphoreType.DMA,
      ],
      compiler_params=pltpu.CompilerParams(use_tc_tiling_on_sc=False),
  )
  def k(table, idx_hbm, out, idx_v, rows_v, tbl_s, sem):
    sid = lax.axis_index("s")
    w = sid * _NC + lax.axis_index("c")

    @pl.when(sid == 0)
    def _():
      pltpu.sync_copy(table, tbl_s)

    pltpu.sync_copy(idx_hbm.at[w], idx_v)
    plsc.subcore_barrier()
    copies = []
    for j in range(per_w):
      copies.append(pltpu.async_copy(
          tbl_s.at[idx_v.at[j]], rows_v.at[pl.ds(j * _CH, _CH)], sem))
    for c in copies:
      c.wait()
    pltpu.sync_copy(rows_v, out.at[pl.ds(w * per_w * _CH, per_w * _CH)])

  return k


def _make_scatter(n_nodes, d, per_w):
  mesh = plsc.VectorSubcoreMesh(core_axis_name="c", subcore_axis_name="s")

  @functools.partial(
      pl.kernel,
      mesh=mesh,
      out_type=jax.ShapeDtypeStruct((_NC, n_nodes, d), jnp.float32),
      scratch_types=[
          pltpu.VMEM((per_w, _CH), jnp.int32),
          pltpu.VMEM((per_w * _CH, d), jnp.float32),
          pltpu.VMEM_SHARED((n_nodes, d), jnp.float32),
      ],
      compiler_params=pltpu.CompilerParams(use_tc_tiling_on_sc=False),
  )
  def k(vals_hbm, idx_hbm, zeros_hbm, out, idx_v, rows_v, acc):
    cid = lax.axis_index("c")
    sid = lax.axis_index("s")
    w = cid * _NS + sid

    @pl.when(sid == 0)
    def _():
      pltpu.sync_copy(zeros_hbm, acc)

    plsc.subcore_barrier()
    pltpu.sync_copy(idx_hbm.at[w], idx_v)
    pltpu.sync_copy(vals_hbm.at[pl.ds(w * per_w * _CH, per_w * _CH)], rows_v)
    for j in range(per_w):
      pltpu.sync_copy(rows_v.at[pl.ds(j * _CH, _CH)],
                      acc.at[idx_v.at[j]], add=True)
    plsc.subcore_barrier()

    @pl.when(sid == 0)
    def _():
      pltpu.sync_copy(acc, out.at[cid])

  return k


def _edge_math_body(g_ref, aux_ref, w_all_ref, w_comp_ref,
                    slw_ref, a_wt_ref, a_b_ref, b_wt_ref, b_b_ref, out_ref):
  f32 = jnp.float32
  gb = g_ref[...]
  aux = aux_ref[...]
  blk = gb.shape[0]
  n_rels, n_bases = w_comp_ref.shape
  d_in = gb.shape[1] // 2
  d_out = slw_ref.shape[1]
  na = a_wt_ref.shape[0] - gb.shape[1]
  relf = aux[:, na:na + 1]
  onehot = (relf == lax.broadcasted_iota(
      jnp.int32, (1, n_rels), 1).astype(f32)).astype(f32)
  coeff = jnp.dot(onehot, w_comp_ref[...], preferred_element_type=f32)
  y = jnp.dot(gb, w_all_ref[...], preferred_element_type=f32)
  msg = jnp.zeros((blk, d_out), dtype=f32)
  for b in range(n_bases):
    msg = msg + coeff[:, b:b + 1] * y[:, b * d_out:(b + 1) * d_out]
  ecat = jnp.concatenate([gb, aux[:, :na]], axis=1)
  h = jnp.maximum(
      jnp.dot(ecat, a_wt_ref[...], preferred_element_type=f32) + a_b_ref[...],
      0.0)
  logit = jnp.dot(h, b_wt_ref[...], preferred_element_type=f32) + b_b_ref[...]
  att = 1.0 / (1.0 + jnp.exp(-logit))
  curr = jnp.dot(gb, slw_ref[...], preferred_element_type=f32) + msg * att
  half = blk // 2
  out_ref[...] = jnp.concatenate([curr[:half], curr[half:]], axis=1)


def _combine_body(p_ref, o_ref):
  o_ref[...] = p_ref[0] + p_ref[1]


def kernel(node_feat, e2n_sp, total_target_relation, total_edge,
           total_relation_embed, total_relation, weight, w_comp,
           self_loop_weight, A_w, A_b, B_w, B_b):
  del e2n_sp
  n_nodes, inp_dim = node_feat.shape
  n_edges = total_edge.shape[1]
  out_dim = self_loop_weight.shape[1]
  attn_dim = total_relation_embed.shape[1]
  n_bases = weight.shape[0]
  f32 = jnp.float32

  grain = _NW * _CH
  e_pad = -(-n_edges // grain) * grain
  pad = e_pad - n_edges
  n_acc = n_nodes + 16

  per_w_g = 2 * e_pad // grain
  idx_int = jnp.stack(
      [jnp.pad(total_edge[0], (0, pad)), jnp.pad(total_edge[1], (0, pad))],
      axis=1).reshape(_NW, per_w_g, _CH)
  g = _make_gather(n_nodes, inp_dim, per_w_g)(node_feat, idx_int)
  gp = g.reshape(e_pad, 2 * inp_dim)

  aux = jnp.concatenate(
      [total_relation_embed, total_target_relation,
       total_relation.astype(f32)[:, None],
       jnp.zeros((n_edges, 2 * inp_dim - 2 * attn_dim - 1), f32)], axis=1)
  aux = jnp.pad(aux, ((0, pad), (0, 0)))

  blk = 4096
  n_blk = e_pad // blk
  w_all = jnp.concatenate(
      [jnp.transpose(weight, (1, 0, 2)).reshape(inp_dim, n_bases * out_dim),
       jnp.zeros((inp_dim, n_bases * out_dim), f32)], axis=0)
  slwext = jnp.concatenate(
      [jnp.zeros((inp_dim, out_dim), f32), self_loop_weight], axis=0)
  curr2 = pl.pallas_call(
      _edge_math_body,
      grid=(n_blk,),
      in_specs=[
          pl.BlockSpec((blk, 2 * inp_dim), lambda i: (i, 0)),
          pl.BlockSpec((blk, 2 * inp_dim), lambda i: (i, 0)),
          pl.BlockSpec(w_all.shape, lambda i: (0, 0)),
          pl.BlockSpec(w_comp.shape, lambda i: (0, 0)),
          pl.BlockSpec(slwext.shape, lambda i: (0, 0)),
          pl.BlockSpec(A_w.shape[::-1], lambda i: (0, 0)),
          pl.BlockSpec((1, A_b.shape[0]), lambda i: (0, 0)),
          pl.BlockSpec(B_w.shape[::-1], lambda i: (0, 0)),
          pl.BlockSpec((1, 1), lambda i: (0, 0)),
      ],
      out_specs=pl.BlockSpec((blk // 2, 2 * out_dim), lambda i: (i, 0)),
      out_shape=jax.ShapeDtypeStruct((e_pad // 2, 2 * out_dim), f32),
  )(gp, aux, w_all, w_comp,
    slwext, A_w.T, A_b.reshape(1, -1), B_w.T, B_b.reshape(1, 1))
  curr = curr2.reshape(e_pad, out_dim)

  per_w_s = e_pad // grain
  idx_t = jnp.pad(total_edge[1], (0, pad), constant_values=n_nodes).reshape(
      e_pad // blk, 2, blk // 2).transpose(0, 2, 1).reshape(
      _NW, per_w_s, _CH)
  partials = _make_scatter(n_acc, out_dim, per_w_s)(
      curr, idx_t, jnp.zeros((n_acc, out_dim), f32))

  summed = pl.pallas_call(
      _combine_body,
      grid=(1,),
      in_specs=[
          pl.BlockSpec((_NC, n_acc // 2, 2 * out_dim), lambda i: (0, 0, 0))],
      out_specs=pl.BlockSpec((n_acc // 2, 2 * out_dim), lambda i: (0, 0)),
      out_shape=jax.ShapeDtypeStruct((n_acc // 2, 2 * out_dim), f32),
  )(partials.reshape(_NC, n_acc // 2, 2 * out_dim))
  return summed.reshape(n_acc, out_dim)[:n_nodes]


# speedup vs baseline: 1.0744x; 1.0626x over previous
import functools

import jax
import jax.numpy as jnp
from jax import lax
from jax.experimental import pallas as pl
from jax.experimental.pallas import tpu as pltpu
from jax.experimental.pallas import tpu_sc as plsc

_NC = 2
_NS = 16
_NW = _NC * _NS
_CH = 128


def _make_gather(n_nodes, d, per_w):
  mesh = plsc.VectorSubcoreMesh(core_axis_name="c", subcore_axis_name="s")

  @functools.partial(
      pl.kernel,
      mesh=mesh,
      out_type=jax.ShapeDtypeStruct((_NW * per_w * _CH, d), jnp.float32),
      scratch_types=[
          pltpu.VMEM((per_w, _CH), jnp.int32),
          pltpu.VMEM((per_w * _CH, d), jnp.float32),
          pltpu.VMEM_SHARED((n_nodes, d), jnp.float32),
          pltpu.SemaphoreType.DMA,
      ],
      compiler_params=pltpu.CompilerParams(use_tc_tiling_on_sc=False),
  )
  def k(table, idx_hbm, out, idx_v, rows_v, tbl_s, sem):
    sid = lax.axis_index("s")
    w = sid * _NC + lax.axis_index("c")

    @pl.when(sid == 0)
    def _():
      pltpu.sync_copy(table, tbl_s)

    pltpu.sync_copy(idx_hbm.at[w], idx_v)
    plsc.subcore_barrier()
    copies = []
    for j in range(per_w):
      copies.append(pltpu.async_copy(
          tbl_s.at[idx_v.at[j]], rows_v.at[pl.ds(j * _CH, _CH)], sem))
    for c in copies:
      c.wait()
    pltpu.sync_copy(rows_v, out.at[pl.ds(w * per_w * _CH, per_w * _CH)])

  return k


def _make_scatter(n_nodes, d, per_w):
  mesh = plsc.VectorSubcoreMesh(core_axis_name="c", subcore_axis_name="s")

  @functools.partial(
      pl.kernel,
      mesh=mesh,
      out_type=jax.ShapeDtypeStruct((_NC, n_nodes, d), jnp.float32),
      scratch_types=[
          pltpu.VMEM((per_w, _CH), jnp.int32),
          pltpu.VMEM((per_w * _CH, d), jnp.float32),
          pltpu.VMEM_SHARED((n_nodes, d), jnp.float32),
      ],
      compiler_params=pltpu.CompilerParams(use_tc_tiling_on_sc=False),
  )
  def k(vals_hbm, idx_hbm, zeros_hbm, out, idx_v, rows_v, acc):
    cid = lax.axis_index("c")
    sid = lax.axis_index("s")
    w = cid * _NS + sid

    @pl.when(sid == 0)
    def _():
      pltpu.sync_copy(zeros_hbm, acc)

    plsc.subcore_barrier()
    pltpu.sync_copy(idx_hbm.at[w], idx_v)
    pltpu.sync_copy(vals_hbm.at[pl.ds(w * per_w * _CH, per_w * _CH)], rows_v)
    for j in range(per_w):
      pltpu.sync_copy(rows_v.at[pl.ds(j * _CH, _CH)],
                      acc.at[idx_v.at[j]], add=True)
    plsc.subcore_barrier()

    @pl.when(sid == 0)
    def _():
      pltpu.sync_copy(acc, out.at[cid])

  return k


def _edge_math_body(g_ref, aux_ref, wext_ref, w_comp_ref,
                    slwext_ref, a_wt_ref, a_b_ref, b_wt_ref, b_b_ref, out_ref):
  f32 = jnp.float32
  gb = g_ref[...]
  aux = aux_ref[...]
  blk = gb.shape[0]
  n_rels, n_bases = w_comp_ref.shape
  na = a_wt_ref.shape[0] - gb.shape[1]
  relf = aux[:, na:na + 1]
  onehot = (relf == lax.broadcasted_iota(
      jnp.int32, (1, n_rels), 1).astype(f32)).astype(f32)
  coeff = jnp.dot(onehot, w_comp_ref[...], preferred_element_type=f32)
  msg = jnp.zeros((blk, out_ref.shape[1] // 2), dtype=f32)
  for b in range(n_bases):
    msg = msg + coeff[:, b:b + 1] * jnp.dot(
        gb, wext_ref[b], preferred_element_type=f32)
  ecat = jnp.concatenate([gb, aux[:, :na]], axis=1)
  h = jnp.maximum(
      jnp.dot(ecat, a_wt_ref[...], preferred_element_type=f32) + a_b_ref[...],
      0.0)
  logit = jnp.dot(h, b_wt_ref[...], preferred_element_type=f32) + b_b_ref[...]
  att = 1.0 / (1.0 + jnp.exp(-logit))
  curr = jnp.dot(gb, slwext_ref[...], preferred_element_type=f32) + msg * att
  half = blk // 2
  out_ref[...] = jnp.concatenate([curr[:half], curr[half:]], axis=1)


def _combine_body(p_ref, o_ref):
  o_ref[...] = p_ref[0] + p_ref[1]


def kernel(node_feat, e2n_sp, total_target_relation, total_edge,
           total_relation_embed, total_relation, weight, w_comp,
           self_loop_weight, A_w, A_b, B_w, B_b):
  del e2n_sp
  n_nodes, inp_dim = node_feat.shape
  n_edges = total_edge.shape[1]
  out_dim = self_loop_weight.shape[1]
  attn_dim = total_relation_embed.shape[1]
  n_bases = weight.shape[0]
  f32 = jnp.float32

  grain = _NW * _CH
  e_pad = -(-n_edges // grain) * grain
  pad = e_pad - n_edges
  n_acc = n_nodes + 16

  per_w_g = 2 * e_pad // grain
  idx_int = jnp.stack(
      [jnp.pad(total_edge[0], (0, pad)), jnp.pad(total_edge[1], (0, pad))],
      axis=1).reshape(_NW, per_w_g, _CH)
  g = _make_gather(n_nodes, inp_dim, per_w_g)(node_feat, idx_int)
  gp = g.reshape(e_pad, 2 * inp_dim)

  aux = jnp.concatenate(
      [total_relation_embed, total_target_relation,
       total_relation.astype(f32)[:, None],
       jnp.zeros((n_edges, 2 * inp_dim - 2 * attn_dim - 1), f32)], axis=1)
  aux = jnp.pad(aux, ((0, pad), (0, 0)))

  blk = 4096
  n_blk = e_pad // blk
  zcol = jnp.zeros((inp_dim, out_dim), f32)
  wext = jnp.concatenate(
      [weight, jnp.broadcast_to(zcol, (n_bases, inp_dim, out_dim))], axis=1)
  slwext = jnp.concatenate([zcol, self_loop_weight], axis=0)
  curr2 = pl.pallas_call(
      _edge_math_body,
      grid=(n_blk,),
      in_specs=[
          pl.BlockSpec((blk, 2 * inp_dim), lambda i: (i, 0)),
          pl.BlockSpec((blk, 2 * inp_dim), lambda i: (i, 0)),
          pl.BlockSpec(wext.shape, lambda i: (0, 0, 0)),
          pl.BlockSpec(w_comp.shape, lambda i: (0, 0)),
          pl.BlockSpec(slwext.shape, lambda i: (0, 0)),
          pl.BlockSpec(A_w.shape[::-1], lambda i: (0, 0)),
          pl.BlockSpec((1, A_b.shape[0]), lambda i: (0, 0)),
          pl.BlockSpec(B_w.shape[::-1], lambda i: (0, 0)),
          pl.BlockSpec((1, 1), lambda i: (0, 0)),
      ],
      out_specs=pl.BlockSpec((blk // 2, 2 * out_dim), lambda i: (i, 0)),
      out_shape=jax.ShapeDtypeStruct((e_pad // 2, 2 * out_dim), f32),
  )(gp, aux, wext, w_comp,
    slwext, A_w.T, A_b.reshape(1, -1), B_w.T, B_b.reshape(1, 1))
  curr = curr2.reshape(e_pad, out_dim)

  per_w_s = e_pad // grain
  idx_t = jnp.pad(total_edge[1], (0, pad), constant_values=n_nodes).reshape(
      e_pad // blk, 2, blk // 2).transpose(0, 2, 1).reshape(
      _NW, per_w_s, _CH)
  partials = _make_scatter(n_acc, out_dim, per_w_s)(
      curr, idx_t, jnp.zeros((n_acc, out_dim), f32))

  summed = pl.pallas_call(
      _combine_body,
      grid=(1,),
      in_specs=[
          pl.BlockSpec((_NC, n_acc // 2, 2 * out_dim), lambda i: (0, 0, 0))],
      out_specs=pl.BlockSpec((n_acc // 2, 2 * out_dim), lambda i: (0, 0)),
      out_shape=jax.ShapeDtypeStruct((n_acc // 2, 2 * out_dim), f32),
  )(partials.reshape(_NC, n_acc // 2, 2 * out_dim))
  return summed.reshape(n_acc, out_dim)[:n_nodes]
